# Initial kernel scaffold; baseline (speedup 1.0000x reference)
import jax
import jax.numpy as jnp
from jax.experimental import pallas as pl


def kernel(feature, edge_index, eps, W1, b1, W2, b2, gamma, beta):
    raise NotImplementedError("write your pallas kernel here")


# speedup vs baseline: 8.1647x; 8.1647x over previous
import functools

import jax
import jax.numpy as jnp
from jax import lax
from jax.experimental import pallas as pl
from jax.experimental.pallas import tpu as pltpu
from jax.experimental.pallas import tpu_sc as plsc

N = 10000
E = 320000
D = 128
HALF = D // 2
NTILES = 16
CHUNK = 80
EPT = E // NTILES
NCHUNK = EPT // CHUNK
ROWS_PER_TILE = N // NTILES


def _sc_segment_sum_plus_x(feature, src2, dst2):
    mesh = plsc.VectorSubcoreMesh(core_axis_name="c", subcore_axis_name="s")

    @functools.partial(
        pl.kernel,
        mesh=mesh,
        compiler_params=pltpu.CompilerParams(use_tc_tiling_on_sc=False),
        out_type=jax.ShapeDtypeStruct((N, D), jnp.float32),
        scratch_types=[
            pltpu.VMEM_SHARED((N, HALF), jnp.float32),
            pltpu.VMEM_SHARED((N, HALF), jnp.float32),
            pltpu.VMEM((NCHUNK, CHUNK), jnp.int32),
            pltpu.VMEM((NCHUNK, CHUNK), jnp.int32),
            pltpu.VMEM((CHUNK, HALF), jnp.float32),
            pltpu.SemaphoreType.DMA,
        ],
    )
    def k(feat_hbm, src_hbm, dst_hbm, out_hbm,
          feat_sh, acc_sh, src_v, dst_v, rows_v, sem):
        cid = lax.axis_index("c")
        sid = lax.axis_index("s")
        r0 = sid * ROWS_PER_TILE
        c0 = cid * HALF
        pltpu.sync_copy(feat_hbm.at[pl.ds(r0, ROWS_PER_TILE), pl.ds(c0, HALF)],
                        feat_sh.at[pl.ds(r0, ROWS_PER_TILE)])
        pltpu.sync_copy(feat_hbm.at[pl.ds(r0, ROWS_PER_TILE), pl.ds(c0, HALF)],
                        acc_sh.at[pl.ds(r0, ROWS_PER_TILE)])
        pltpu.sync_copy(src_hbm.at[pl.ds(sid * NCHUNK, NCHUNK)], src_v)
        pltpu.sync_copy(dst_hbm.at[pl.ds(sid * NCHUNK, NCHUNK)], dst_v)
        plsc.subcore_barrier()

        def body(j, carry):
            pltpu.async_copy(feat_sh.at[src_v.at[j]], rows_v, sem).wait()
            pltpu.sync_copy(rows_v, acc_sh.at[dst_v.at[j]], add=True)
            return carry

        lax.fori_loop(0, NCHUNK, body, 0)
        plsc.subcore_barrier()
        pltpu.sync_copy(acc_sh.at[pl.ds(r0, ROWS_PER_TILE)],
                        out_hbm.at[pl.ds(r0, ROWS_PER_TILE), pl.ds(c0, HALF)])

    return k(feature, src2, dst2)


def _tc_mlp_bn(pooled_plus_x, feature, eps, W1, b1, W2, b2, gamma, beta):
    def body(eps_ref, pp_ref, x_ref, w1_ref, b1_ref, w2_ref, b2_ref,
             g_ref, bt_ref, o_ref):
        y = pp_ref[...] + eps_ref[0] * x_ref[...]
        h = jnp.dot(y, w1_ref[...], preferred_element_type=jnp.float32)
        h = jnp.maximum(h + b1_ref[...], 0.0)
        h = jnp.dot(h, w2_ref[...], preferred_element_type=jnp.float32)
        h = h + b2_ref[...]
        mean = jnp.mean(h, axis=0, keepdims=True)
        d = h - mean
        var = jnp.mean(d * d, axis=0, keepdims=True)
        h = d * lax.rsqrt(var + 1e-5) * g_ref[...] + bt_ref[...]
        o_ref[...] = jnp.maximum(h, 0.0)

    vspec = pl.BlockSpec(memory_space=pltpu.VMEM)
    return pl.pallas_call(
        body,
        out_shape=jax.ShapeDtypeStruct((N, D), jnp.float32),
        in_specs=[pl.BlockSpec(memory_space=pltpu.SMEM)] + [vspec] * 8,
        out_specs=vspec,
    )(eps, pooled_plus_x, feature, W1, b1.reshape(1, D), W2,
      b2.reshape(1, D), gamma.reshape(1, D), beta.reshape(1, D))


def kernel(feature, edge_index, eps, W1, b1, W2, b2, gamma, beta):
    src2 = edge_index[0].reshape(E // CHUNK, CHUNK)
    dst2 = edge_index[1].reshape(E // CHUNK, CHUNK)
    pooled_plus_x = _sc_segment_sum_plus_x(feature, src2, dst2)
    return _tc_mlp_bn(pooled_plus_x, feature, eps, W1, b1, W2, b2,
                      gamma, beta)
